# baseline (speedup 1.0000x reference)
import functools

import jax
import jax.numpy as jnp
from jax import lax
from jax.experimental import pallas as pl
from jax.experimental.pallas import tpu as pltpu
from jax.experimental.pallas import tpu_sc as plsc

BATCH = 16384
EMB = 64
BERT_GENRE = 786
HID = 256

NC = 2
NS = 16
NW = NC * NS
BPW = BATCH // NW
GRP = 32
NGRP = BPW // GRP
NSLOT = 8
DEPTH = NSLOT - 1


def _user_gather_body(idx_hbm, tab3_hbm, out_hbm, idxs_v, blk_v, oute_v, *sems):
    wid = lax.axis_index("s") * NC + lax.axis_index("c")
    base = wid * BPW

    def stage(row0):
        pltpu.sync_copy(idx_hbm.at[pl.ds(row0, GRP)], idxs_v)
        return tuple(idxs_v[pl.ds(16 * h, 16)] for h in range(GRP // 16))

    def sidx(vgs, j):
        return vgs[j // 16][j % 16]

    def fire(vgs, j):
        b0 = lax.shift_right_logical(sidx(vgs, j), 7) * 128
        for a in range(8):
            pltpu.async_copy(tab3_hbm.at[a, :, pl.ds(b0, 128)],
                             blk_v.at[j % NSLOT, a], sems[j % NSLOT])

    def drain(j):
        pltpu.make_async_copy(tab3_hbm.at[:, :, pl.ds(0, 128)],
                              blk_v.at[j % NSLOT], sems[j % NSLOT]).wait()

    def extract(vgs, j):
        lanev = jnp.full((16,), jnp.bitwise_and(sidx(vgs, j), 127), jnp.int32)
        for g2 in range(EMB // 16):
            cv = lax.iota(jnp.int32, 16) + 16 * g2
            av = lax.shift_right_logical(cv, 3)
            sv = jnp.bitwise_and(cv, 7)
            vals = plsc.load_gather(blk_v.at[j % NSLOT], [av, sv, lanev])
            oute_v[j, pl.ds(16 * g2, 16)] = vals

    def body(g, vgs):
        nvgs = lax.cond(g + 1 < NGRP,
                        lambda: stage(base + (g + 1) * GRP),
                        lambda: vgs)
        for j in range(GRP):
            drain(j)
            extract(vgs, j)
            nj = j + DEPTH
            if nj < GRP:
                fire(vgs, nj)
            else:
                @pl.when(g + 1 < NGRP)
                def _():
                    fire(nvgs, nj - GRP)
        pltpu.sync_copy(oute_v, out_hbm.at[pl.ds(base + g * GRP, GRP)])
        return nvgs

    vgs0 = stage(base)
    for j in range(DEPTH):
        fire(vgs0, j)
    lax.fori_loop(0, NGRP, body, vgs0)


@jax.jit
def _sc_user_gather(user_idx, utab3):
    mesh = plsc.VectorSubcoreMesh(core_axis_name="c", subcore_axis_name="s")
    f = functools.partial(
        pl.kernel,
        mesh=mesh,
        compiler_params=pltpu.CompilerParams(use_tc_tiling_on_sc=True,
                                             needs_layout_passes=False),
        out_type=jax.ShapeDtypeStruct((BATCH, EMB), jnp.float32),
        scratch_types=[
            pltpu.VMEM((GRP,), jnp.int32),
            pltpu.VMEM((NSLOT, 8, 8, 128), jnp.float32),
            pltpu.VMEM((GRP, EMB), jnp.float32),
        ] + [pltpu.SemaphoreType.DMA] * NSLOT,
    )(_user_gather_body)
    return f(user_idx, utab3)


MCH = 64
NMCH = BPW // MCH


def _movie_gather_body(idx_hbm, tab_hbm, out_hbm, idxc_v, oute_v, sem):
    wid = lax.axis_index("s") * NC + lax.axis_index("c")
    base = wid * BPW

    def chunk_body(k, _):
        row0 = base + k * MCH
        pltpu.sync_copy(idx_hbm.at[pl.ds(row0, MCH)], idxc_v)
        for g in range(MCH // 16):
            vg = idxc_v[pl.ds(16 * g, 16)]
            for j in range(16):
                pltpu.async_copy(tab_hbm.at[vg[j]], oute_v.at[16 * g + j], sem)
        for j in range(MCH):
            pltpu.make_async_copy(tab_hbm.at[0], oute_v.at[j], sem).wait()
        pltpu.sync_copy(oute_v, out_hbm.at[pl.ds(row0, MCH)])
        return ()

    lax.fori_loop(0, NMCH, chunk_body, ())


@jax.jit
def _sc_movie_gather(movie_idx, movie_table):
    mesh = plsc.VectorSubcoreMesh(core_axis_name="c", subcore_axis_name="s")
    f = functools.partial(
        pl.kernel,
        mesh=mesh,
        compiler_params=pltpu.CompilerParams(use_tc_tiling_on_sc=True,
                                             needs_layout_passes=False),
        out_type=jax.ShapeDtypeStruct((BATCH, EMB), jnp.float32),
        scratch_types=[
            pltpu.VMEM((MCH,), jnp.int32),
            pltpu.VMEM((MCH, EMB), jnp.float32),
            pltpu.SemaphoreType.DMA,
        ],
    )(_movie_gather_body)
    return f(movie_idx, movie_table)


BB = 1024


def _mlp1_body(mfvt_ref, w1f_ref, b1_ref, p_ref):
    p_ref[...] = lax.dot_general(
        mfvt_ref[...], w1f_ref[...], (((0,), (0,)), ((), ())),
        preferred_element_type=jnp.float32) + b1_ref[...]


@jax.jit
def _mlp_partial(mfvt, w1f, b1):
    nblk = BATCH // BB
    return pl.pallas_call(
        _mlp1_body,
        grid=(nblk,),
        in_specs=[
            pl.BlockSpec((BERT_GENRE, BB), lambda i: (0, i)),
            pl.BlockSpec((BERT_GENRE, HID), lambda i: (0, 0)),
            pl.BlockSpec((1, HID), lambda i: (0, 0)),
        ],
        out_specs=pl.BlockSpec((BB, HID), lambda i: (i, 0)),
        out_shape=jax.ShapeDtypeStruct((BATCH, HID), jnp.float32),
    )(mfvt, w1f, b1)


def _mlp2_body(p_ref, ue_ref, me_ref, w1u_ref, w1m_ref,
               w2_ref, b2_ref, w3_ref, b3_ref, out_ref):
    h = p_ref[...] + ue_ref[...] @ w1u_ref[...] + me_ref[...] @ w1m_ref[...]
    h = jnp.maximum(h, 0.0)
    h = jnp.maximum(h @ w2_ref[...] + b2_ref[...], 0.0)
    o = h @ w3_ref[...] + b3_ref[...]
    out_ref[...] = jax.nn.sigmoid(o)[:, 0]


@jax.jit
def _mlp_final(p, ue, me, w1u, w1m, w2, b2, w3, b3):
    nblk = BATCH // BB
    return pl.pallas_call(
        _mlp2_body,
        grid=(nblk,),
        in_specs=[
            pl.BlockSpec((BB, HID), lambda i: (i, 0)),
            pl.BlockSpec((BB, EMB), lambda i: (i, 0)),
            pl.BlockSpec((BB, EMB), lambda i: (i, 0)),
            pl.BlockSpec((EMB, HID), lambda i: (0, 0)),
            pl.BlockSpec((EMB, HID), lambda i: (0, 0)),
            pl.BlockSpec((HID, HID // 2), lambda i: (0, 0)),
            pl.BlockSpec((1, HID // 2), lambda i: (0, 0)),
            pl.BlockSpec((HID // 2, 1), lambda i: (0, 0)),
            pl.BlockSpec((1, 1), lambda i: (0, 0)),
        ],
        out_specs=pl.BlockSpec((BB,), lambda i: (i,)),
        out_shape=jax.ShapeDtypeStruct((BATCH,), jnp.float32),
    )(p, ue, me, w1u, w1m, w2, b2, w3, b3)


def kernel(user, movie, movie_feature_vec, user_table, movie_table,
           W1, b1, W2, b2, W3, b3):
    utab3 = user_table.T.reshape(8, 8, user_table.shape[0])
    ue = _sc_user_gather(user, utab3)
    p = _mlp_partial(movie_feature_vec.T, W1[2 * EMB:], b1[None, :])
    movie_q, ue, p = lax.optimization_barrier((movie, ue, p))
    me = _sc_movie_gather(movie_q, movie_table)
    return _mlp_final(p, ue, me, W1[:EMB], W1[EMB:2 * EMB],
                      W2, b2[None, :], W3, b3[None, :])


# speedup vs baseline: 1.0176x; 1.0176x over previous
import functools

import jax
import jax.numpy as jnp
from jax import lax
from jax.experimental import pallas as pl
from jax.experimental.pallas import tpu as pltpu
from jax.experimental.pallas import tpu_sc as plsc

BATCH = 16384
EMB = 64
BERT_GENRE = 786
HID = 256

NC = 2
NS = 16
NW = NC * NS
BPW = BATCH // NW
GRP = 32
NGRP = BPW // GRP
NSLOT = 8
DEPTH = NSLOT - 1


def _user_gather_body(idx_hbm, tab3_hbm, out_hbm, idxs_v, blk_v, oute_v, *sems):
    wid = lax.axis_index("s") * NC + lax.axis_index("c")
    base = wid * BPW

    def stage(row0):
        pltpu.sync_copy(idx_hbm.at[pl.ds(row0, GRP)], idxs_v)
        return tuple(idxs_v[pl.ds(16 * h, 16)] for h in range(GRP // 16))

    def sidx(vgs, j):
        return vgs[j // 16][j % 16]

    def fire(vgs, j):
        b0 = lax.shift_right_logical(sidx(vgs, j), 7) * 128
        pltpu.async_copy(tab3_hbm.at[:, :, pl.ds(b0, 128)],
                         blk_v.at[j % NSLOT], sems[j % NSLOT])

    def drain(j):
        pltpu.make_async_copy(tab3_hbm.at[:, :, pl.ds(0, 128)],
                              blk_v.at[j % NSLOT], sems[j % NSLOT]).wait()

    def extract(vgs, j):
        lanev = jnp.full((16,), jnp.bitwise_and(sidx(vgs, j), 127), jnp.int32)
        for g2 in range(EMB // 16):
            cv = lax.iota(jnp.int32, 16) + 16 * g2
            av = lax.shift_right_logical(cv, 3)
            sv = jnp.bitwise_and(cv, 7)
            vals = plsc.load_gather(blk_v.at[j % NSLOT], [av, sv, lanev])
            oute_v[j, pl.ds(16 * g2, 16)] = vals

    def body(g, vgs):
        nvgs = lax.cond(g + 1 < NGRP,
                        lambda: stage(base + (g + 1) * GRP),
                        lambda: vgs)
        for j in range(GRP):
            drain(j)
            extract(vgs, j)
            nj = j + DEPTH
            if nj < GRP:
                fire(vgs, nj)
            else:
                @pl.when(g + 1 < NGRP)
                def _():
                    fire(nvgs, nj - GRP)
        pltpu.sync_copy(oute_v, out_hbm.at[pl.ds(base + g * GRP, GRP)])
        return nvgs

    vgs0 = stage(base)
    for j in range(DEPTH):
        fire(vgs0, j)
    lax.fori_loop(0, NGRP, body, vgs0)


@jax.jit
def _sc_user_gather(user_idx, utab3):
    mesh = plsc.VectorSubcoreMesh(core_axis_name="c", subcore_axis_name="s")
    f = functools.partial(
        pl.kernel,
        mesh=mesh,
        compiler_params=pltpu.CompilerParams(use_tc_tiling_on_sc=True,
                                             needs_layout_passes=False),
        out_type=jax.ShapeDtypeStruct((BATCH, EMB), jnp.float32),
        scratch_types=[
            pltpu.VMEM((GRP,), jnp.int32),
            pltpu.VMEM((NSLOT, 8, 8, 128), jnp.float32),
            pltpu.VMEM((GRP, EMB), jnp.float32),
        ] + [pltpu.SemaphoreType.DMA] * NSLOT,
    )(_user_gather_body)
    return f(user_idx, utab3)


MCH = 64
NMCH = BPW // MCH


def _movie_gather_body(idx_hbm, tab_hbm, out_hbm, idxc_v, oute_v, sem):
    wid = lax.axis_index("s") * NC + lax.axis_index("c")
    base = wid * BPW

    def chunk_body(k, _):
        row0 = base + k * MCH
        pltpu.sync_copy(idx_hbm.at[pl.ds(row0, MCH)], idxc_v)
        for g in range(MCH // 16):
            vg = idxc_v[pl.ds(16 * g, 16)]
            for j in range(16):
                pltpu.async_copy(tab_hbm.at[vg[j]], oute_v.at[16 * g + j], sem)
        for j in range(MCH):
            pltpu.make_async_copy(tab_hbm.at[0], oute_v.at[j], sem).wait()
        pltpu.sync_copy(oute_v, out_hbm.at[pl.ds(row0, MCH)])
        return ()

    lax.fori_loop(0, NMCH, chunk_body, ())


@jax.jit
def _sc_movie_gather(movie_idx, movie_table):
    mesh = plsc.VectorSubcoreMesh(core_axis_name="c", subcore_axis_name="s")
    f = functools.partial(
        pl.kernel,
        mesh=mesh,
        compiler_params=pltpu.CompilerParams(use_tc_tiling_on_sc=True,
                                             needs_layout_passes=False),
        out_type=jax.ShapeDtypeStruct((BATCH, EMB), jnp.float32),
        scratch_types=[
            pltpu.VMEM((MCH,), jnp.int32),
            pltpu.VMEM((MCH, EMB), jnp.float32),
            pltpu.SemaphoreType.DMA,
        ],
    )(_movie_gather_body)
    return f(movie_idx, movie_table)


BB = 2048


def _mlp1_body(mfvt_ref, w1f_ref, b1_ref, p_ref):
    p_ref[...] = lax.dot_general(
        mfvt_ref[...], w1f_ref[...], (((0,), (0,)), ((), ())),
        preferred_element_type=jnp.float32) + b1_ref[...]


@jax.jit
def _mlp_partial(mfvt, w1f, b1):
    nblk = BATCH // BB
    return pl.pallas_call(
        _mlp1_body,
        grid=(nblk,),
        in_specs=[
            pl.BlockSpec((BERT_GENRE, BB), lambda i: (0, i)),
            pl.BlockSpec((BERT_GENRE, HID), lambda i: (0, 0)),
            pl.BlockSpec((1, HID), lambda i: (0, 0)),
        ],
        out_specs=pl.BlockSpec((BB, HID), lambda i: (i, 0)),
        out_shape=jax.ShapeDtypeStruct((BATCH, HID), jnp.float32),
    )(mfvt, w1f, b1)


def _mlp2_body(p_ref, ue_ref, me_ref, w1u_ref, w1m_ref,
               w2_ref, b2_ref, w3_ref, b3_ref, out_ref):
    h = p_ref[...] + ue_ref[...] @ w1u_ref[...] + me_ref[...] @ w1m_ref[...]
    h = jnp.maximum(h, 0.0)
    h = jnp.maximum(h @ w2_ref[...] + b2_ref[...], 0.0)
    o = h @ w3_ref[...] + b3_ref[...]
    out_ref[...] = jax.nn.sigmoid(o)[:, 0]


@jax.jit
def _mlp_final(p, ue, me, w1u, w1m, w2, b2, w3, b3):
    nblk = BATCH // BB
    return pl.pallas_call(
        _mlp2_body,
        grid=(nblk,),
        in_specs=[
            pl.BlockSpec((BB, HID), lambda i: (i, 0)),
            pl.BlockSpec((BB, EMB), lambda i: (i, 0)),
            pl.BlockSpec((BB, EMB), lambda i: (i, 0)),
            pl.BlockSpec((EMB, HID), lambda i: (0, 0)),
            pl.BlockSpec((EMB, HID), lambda i: (0, 0)),
            pl.BlockSpec((HID, HID // 2), lambda i: (0, 0)),
            pl.BlockSpec((1, HID // 2), lambda i: (0, 0)),
            pl.BlockSpec((HID // 2, 1), lambda i: (0, 0)),
            pl.BlockSpec((1, 1), lambda i: (0, 0)),
        ],
        out_specs=pl.BlockSpec((BB,), lambda i: (i,)),
        out_shape=jax.ShapeDtypeStruct((BATCH,), jnp.float32),
    )(p, ue, me, w1u, w1m, w2, b2, w3, b3)


def kernel(user, movie, movie_feature_vec, user_table, movie_table,
           W1, b1, W2, b2, W3, b3):
    utab3 = user_table.T.reshape(8, 8, user_table.shape[0])
    ue = _sc_user_gather(user, utab3)
    p = _mlp_partial(movie_feature_vec.T, W1[2 * EMB:], b1[None, :])
    movie_q, ue, p = lax.optimization_barrier((movie, ue, p))
    me = _sc_movie_gather(movie_q, movie_table)
    return _mlp_final(p, ue, me, W1[:EMB], W1[EMB:2 * EMB],
                      W2, b2[None, :], W3, b3[None, :])
